# initial kernel scaffold (unmeasured)
import jax
import jax.numpy as jnp
from jax import lax
from jax.experimental import pallas as pl
from jax.experimental.pallas import tpu as pltpu


def kernel(
    x,
):
    def body(*refs):
        pass

    out_shape = jax.ShapeDtypeStruct(..., jnp.float32)
    return pl.pallas_call(body, out_shape=out_shape)(...)


# baseline (device time: 27290 ns/iter reference)
import jax
import jax.numpy as jnp
from jax import lax
from jax.experimental import pallas as pl
from jax.experimental.pallas import tpu as pltpu

N_DEV = 16


def kernel(x):
    m, n = x.shape

    def body(x_ref, out_ref, send_sems, recv_sems):
        my = lax.axis_index("i")

        barrier = pltpu.get_barrier_semaphore()
        for k in range(1, N_DEV):
            pl.semaphore_signal(
                barrier,
                inc=1,
                device_id=((my + k) % N_DEV,),
                device_id_type=pl.DeviceIdType.MESH,
            )
        pl.semaphore_wait(barrier, N_DEV - 1)

        out_ref[my] = x_ref[...].astype(jnp.bfloat16)

        for k in range(1, N_DEV):
            rdma = pltpu.make_async_remote_copy(
                src_ref=out_ref.at[my],
                dst_ref=out_ref.at[my],
                send_sem=send_sems.at[k - 1],
                recv_sem=recv_sems.at[k - 1],
                device_id=((my + k) % N_DEV,),
                device_id_type=pl.DeviceIdType.MESH,
            )
            rdma.start()

        for k in range(1, N_DEV):
            recv = pltpu.make_async_remote_copy(
                src_ref=out_ref.at[my],
                dst_ref=out_ref.at[(my - k) % N_DEV],
                send_sem=send_sems.at[k - 1],
                recv_sem=recv_sems.at[k - 1],
                device_id=((my + k) % N_DEV,),
                device_id_type=pl.DeviceIdType.MESH,
            )
            recv.wait_recv()

        for k in range(1, N_DEV):
            send = pltpu.make_async_remote_copy(
                src_ref=out_ref.at[my],
                dst_ref=out_ref.at[my],
                send_sem=send_sems.at[k - 1],
                recv_sem=recv_sems.at[k - 1],
                device_id=((my + k) % N_DEV,),
                device_id_type=pl.DeviceIdType.MESH,
            )
            send.wait_send()

    out = pl.pallas_call(
        body,
        out_shape=jax.ShapeDtypeStruct((N_DEV, m, n), jnp.bfloat16),
        in_specs=[pl.BlockSpec(memory_space=pltpu.VMEM)],
        out_specs=pl.BlockSpec(memory_space=pltpu.VMEM),
        scratch_shapes=[
            pltpu.SemaphoreType.DMA((N_DEV - 1,)),
            pltpu.SemaphoreType.DMA((N_DEV - 1,)),
        ],
        compiler_params=pltpu.CompilerParams(collective_id=0),
    )(x)
    return out.reshape(N_DEV * m, n)


# device time: 21214 ns/iter; 1.2864x vs baseline; 1.2864x over previous
import jax
import jax.numpy as jnp
from jax import lax
from jax.experimental import pallas as pl
from jax.experimental.pallas import tpu as pltpu

N_DEV = 16
SIDE = 8


def kernel(x):
    m, n = x.shape

    def body(x_ref, out_ref, own_send, own_recv, rel_send, rel_recv,
             pex_send, pex_recv):
        my = lax.axis_index("i")
        side_base = (my // SIDE) * SIDE
        w = my % SIDE
        partner = 4 * (3 - my // 4) + my % 4

        def in_side_peer(t):
            return side_base + (w + t) % SIDE

        def in_side_src(t):
            return side_base + (w - t) % SIDE

        barrier = pltpu.get_barrier_semaphore()
        for t in range(1, SIDE):
            pl.semaphore_signal(
                barrier, inc=1,
                device_id=(in_side_peer(t),),
                device_id_type=pl.DeviceIdType.MESH,
            )
        pl.semaphore_signal(
            barrier, inc=1,
            device_id=(partner,),
            device_id_type=pl.DeviceIdType.MESH,
        )
        pl.semaphore_wait(barrier, SIDE)

        out_ref[my] = x_ref[...].astype(jnp.bfloat16)

        pex = pltpu.make_async_remote_copy(
            src_ref=out_ref.at[my],
            dst_ref=out_ref.at[my],
            send_sem=pex_send.at[0],
            recv_sem=pex_recv.at[0],
            device_id=(partner,),
            device_id_type=pl.DeviceIdType.MESH,
        )
        pex.start()

        for t in range(1, SIDE):
            rdma = pltpu.make_async_remote_copy(
                src_ref=out_ref.at[my],
                dst_ref=out_ref.at[my],
                send_sem=own_send.at[t - 1],
                recv_sem=own_recv.at[t - 1],
                device_id=(in_side_peer(t),),
                device_id_type=pl.DeviceIdType.MESH,
            )
            rdma.start()

        pex_w = pltpu.make_async_remote_copy(
            src_ref=out_ref.at[my],
            dst_ref=out_ref.at[partner],
            send_sem=pex_send.at[0],
            recv_sem=pex_recv.at[0],
            device_id=(partner,),
            device_id_type=pl.DeviceIdType.MESH,
        )
        pex_w.wait_recv()

        for t in range(1, SIDE):
            rdma = pltpu.make_async_remote_copy(
                src_ref=out_ref.at[partner],
                dst_ref=out_ref.at[partner],
                send_sem=rel_send.at[t - 1],
                recv_sem=rel_recv.at[t - 1],
                device_id=(in_side_peer(t),),
                device_id_type=pl.DeviceIdType.MESH,
            )
            rdma.start()

        for t in range(1, SIDE):
            src = in_side_src(t)
            recv = pltpu.make_async_remote_copy(
                src_ref=out_ref.at[my],
                dst_ref=out_ref.at[src],
                send_sem=own_send.at[t - 1],
                recv_sem=own_recv.at[t - 1],
                device_id=(in_side_peer(t),),
                device_id_type=pl.DeviceIdType.MESH,
            )
            recv.wait_recv()
        for t in range(1, SIDE):
            src = in_side_src(t)
            src_partner = 4 * (3 - src // 4) + src % 4
            recv = pltpu.make_async_remote_copy(
                src_ref=out_ref.at[my],
                dst_ref=out_ref.at[src_partner],
                send_sem=rel_send.at[t - 1],
                recv_sem=rel_recv.at[t - 1],
                device_id=(in_side_peer(t),),
                device_id_type=pl.DeviceIdType.MESH,
            )
            recv.wait_recv()

        pex_w.wait_send()
        for sems in (own_send, rel_send):
            for t in range(1, SIDE):
                send = pltpu.make_async_remote_copy(
                    src_ref=out_ref.at[my],
                    dst_ref=out_ref.at[my],
                    send_sem=sems.at[t - 1],
                    recv_sem=own_recv.at[t - 1],
                    device_id=(in_side_peer(t),),
                    device_id_type=pl.DeviceIdType.MESH,
                )
                send.wait_send()

    out = pl.pallas_call(
        body,
        out_shape=jax.ShapeDtypeStruct((N_DEV, m, n), jnp.bfloat16),
        in_specs=[pl.BlockSpec(memory_space=pltpu.VMEM)],
        out_specs=pl.BlockSpec(memory_space=pltpu.VMEM),
        scratch_shapes=[
            pltpu.SemaphoreType.DMA((SIDE - 1,)),
            pltpu.SemaphoreType.DMA((SIDE - 1,)),
            pltpu.SemaphoreType.DMA((SIDE - 1,)),
            pltpu.SemaphoreType.DMA((SIDE - 1,)),
            pltpu.SemaphoreType.DMA((1,)),
            pltpu.SemaphoreType.DMA((1,)),
        ],
        compiler_params=pltpu.CompilerParams(collective_id=0),
    )(x)
    return out.reshape(N_DEV * m, n)
